# baseline (speedup 1.0000x reference)
import functools

import jax
import jax.numpy as jnp
from jax import lax
from jax.experimental import pallas as pl
from jax.experimental.pallas import tpu as pltpu
from jax.experimental.pallas import tpu_sc as plsc

_BLK = 8
_KEEP = 4

_N = 4096
_NW = 32
_NV = _N // 16

_SC_ROWS = 256
_TC_TM = 256
_TC_TN = 128


def _key2d(s):
    b = jax.lax.bitcast_convert_type(jnp.where(s == 0.0, 0.0, s), jnp.int32)
    return b ^ jax.lax.shift_right_logical(jax.lax.shift_right_arithmetic(b, 31), 1)


def _wgroll(a, k):
    n = a.shape[-1]
    idx = (jnp.arange(n) // _BLK) * _BLK + (jnp.arange(n) - k) % _BLK
    idx = jnp.broadcast_to(idx[None, :], a.shape)
    return jnp.take_along_axis(a, idx, axis=-1)


def _tc_kernel_body(x_ref, s_ref, o_ref):
    s = s_ref[...]
    x = x_ref[...]
    key = _key2d(s)
    key1 = key + 1
    lane = jax.lax.broadcasted_iota(jnp.int32, s.shape, 1) % _BLK
    rank = jnp.zeros(s.shape, jnp.float32)
    for k in range(1, 5):
        t = _wgroll(key, k)
        c = t < jnp.where(lane >= k, key1, key)
        cf = jnp.where(c, 1.0, 0.0)
        rank = rank + cf
        if k < 4:
            rank = rank - _wgroll(cf, -k)
    o_ref[...] = jnp.where(rank >= 1.0, x, 0.0)


def _tc_call(x, score, rows):
    m, n = x.shape
    grid = (rows // _TC_TM, n // _TC_TN)
    spec = pl.BlockSpec((_TC_TM, _TC_TN), lambda i, j: (i, j))
    return pl.pallas_call(
        _tc_kernel_body,
        grid=grid,
        in_specs=[spec, spec],
        out_specs=spec,
        out_shape=jax.ShapeDtypeStruct((m, n), x.dtype),
        compiler_params=pltpu.CompilerParams(
            dimension_semantics=("parallel", "parallel"),
        ),
    )(x, score)


def _sc_body(row0, rpw, x_hbm, s_hbm, o_hbm, sbuf, xbuf, obuf):
    cid = lax.axis_index("c")
    sid = lax.axis_index("s")
    wid = sid * 2 + cid

    i16 = lax.iota(jnp.int32, 16)
    lane = i16 & (_BLK - 1)
    group = i16 & ~(_BLK - 1)
    perms = [group + ((i16 - k) & (_BLK - 1)) for k in range(1, _BLK)]
    padds = [jnp.where(lane >= k, 1, 0) for k in range(1, _BLK)]

    def row_loop(r, carry):
        orow = wid * rpw + r
        irow = row0 + orow
        pltpu.sync_copy(s_hbm.at[irow], sbuf)
        pltpu.sync_copy(x_hbm.at[irow], xbuf)

        def vec_loop(j, c2):
            base = j * 16
            key = _key2d(sbuf[pl.ds(base, 16)])
            rank = jnp.zeros((16,), jnp.int32)
            for k in range(1, _BLK):
                t = key.at[perms[k - 1]].get(mode="promise_in_bounds")
                ck = t < key + padds[k - 1]
                rank = rank + jnp.where(ck, 1, 0)
            xv = xbuf[pl.ds(base, 16)]
            obuf[pl.ds(base, 16)] = jnp.where(rank >= _BLK - _KEEP, xv, 0.0)
            return c2

        lax.fori_loop(0, _NV, vec_loop, 0)
        pltpu.sync_copy(obuf, o_hbm.at[orow])
        return carry

    lax.fori_loop(0, rpw, row_loop, 0)


def _sc_call(x, score, row0, rows):
    n = x.shape[1]
    mesh = plsc.VectorSubcoreMesh(core_axis_name="c", subcore_axis_name="s")
    f = functools.partial(
        pl.kernel,
        mesh=mesh,
        out_type=jax.ShapeDtypeStruct((rows, n), jnp.float32),
        scratch_types=[
            pltpu.VMEM((n,), jnp.float32),
            pltpu.VMEM((n,), jnp.float32),
            pltpu.VMEM((n,), jnp.float32),
        ],
    )(functools.partial(_sc_body, row0, rows // _NW))
    return f(x, score)


def kernel(x, score):
    m = x.shape[0]
    r = m - _SC_ROWS
    z = _sc_call(x, score, r, _SC_ROWS)
    y = _tc_call(x, score, r)
    return lax.dynamic_update_slice(y, z, (r, 0))


# speedup vs baseline: 3.2900x; 3.2900x over previous
import functools

import jax
import jax.numpy as jnp
from jax import lax
from jax.experimental import pallas as pl
from jax.experimental.pallas import tpu as pltpu
from jax.experimental.pallas import tpu_sc as plsc

_BLK = 8
_KEEP = 4

_N = 4096
_NW = 32
_NV = _N // 16

_SC_ROWS = 512
_TC_TM = 3840
_TC_TN = 128


def _key2d(s):
    b = jax.lax.bitcast_convert_type(jnp.where(s == 0.0, 0.0, s), jnp.int32)
    return b ^ jax.lax.shift_right_logical(jax.lax.shift_right_arithmetic(b, 31), 1)


def _wgroll(a, k):
    n = a.shape[-1]
    idx = (jnp.arange(n) // _BLK) * _BLK + (jnp.arange(n) - k) % _BLK
    idx = jnp.broadcast_to(idx[None, :], a.shape)
    return jnp.take_along_axis(a, idx, axis=-1)


def _tc_kernel_body(x_ref, s_ref, o_ref):
    s = s_ref[...]
    x = x_ref[...]
    key = _key2d(s)
    key1 = key + 1
    lane = jax.lax.broadcasted_iota(jnp.int32, s.shape, 1) % _BLK
    rank = jnp.zeros(s.shape, jnp.float32)
    for k in range(1, 5):
        t = _wgroll(key, k)
        c = t < jnp.where(lane >= k, key1, key)
        cf = jnp.where(c, 1.0, 0.0)
        rank = rank + cf
        if k < 4:
            rank = rank - _wgroll(cf, -k)
    o_ref[...] = jnp.where(rank >= 1.0, x, 0.0)


def _tc_call(x, score, rows):
    m, n = x.shape
    grid = (rows // _TC_TM, n // _TC_TN)
    spec = pl.BlockSpec((_TC_TM, _TC_TN), lambda i, j: (i, j))
    return pl.pallas_call(
        _tc_kernel_body,
        grid=grid,
        in_specs=[spec, spec],
        out_specs=spec,
        out_shape=jax.ShapeDtypeStruct((m, n), x.dtype),
        compiler_params=pltpu.CompilerParams(
            dimension_semantics=("parallel", "parallel"),
        ),
    )(x, score)


def _sc_body(row0, rpw, x_hbm, s_hbm, o_hbm, sbuf, xbuf, obuf):
    cid = lax.axis_index("c")
    sid = lax.axis_index("s")
    wid = sid * 2 + cid

    i16 = lax.iota(jnp.int32, 16)
    lane = i16 & (_BLK - 1)
    group = i16 & ~(_BLK - 1)
    perms = [group + ((i16 - k) & (_BLK - 1)) for k in range(1, _BLK)]
    padds = [jnp.where(lane >= k, 1, 0) for k in range(1, _BLK)]

    def row_loop(r, carry):
        orow = wid * rpw + r
        irow = row0 + orow
        pltpu.sync_copy(s_hbm.at[irow], sbuf)
        pltpu.sync_copy(x_hbm.at[irow], xbuf)

        def vec_loop(j, c2):
            base = j * 16
            key = _key2d(sbuf[pl.ds(base, 16)])
            rank = jnp.zeros((16,), jnp.int32)
            for k in range(1, _BLK):
                t = key.at[perms[k - 1]].get(mode="promise_in_bounds")
                ck = t < key + padds[k - 1]
                rank = rank + jnp.where(ck, 1, 0)
            xv = xbuf[pl.ds(base, 16)]
            obuf[pl.ds(base, 16)] = jnp.where(rank >= _BLK - _KEEP, xv, 0.0)
            return c2

        lax.fori_loop(0, _NV, vec_loop, 0)
        pltpu.sync_copy(obuf, o_hbm.at[orow])
        return carry

    lax.fori_loop(0, rpw, row_loop, 0)


def _sc_call(x, score, row0, rows):
    n = x.shape[1]
    mesh = plsc.VectorSubcoreMesh(core_axis_name="c", subcore_axis_name="s")
    f = functools.partial(
        pl.kernel,
        mesh=mesh,
        out_type=jax.ShapeDtypeStruct((rows, n), jnp.float32),
        scratch_types=[
            pltpu.VMEM((n,), jnp.float32),
            pltpu.VMEM((n,), jnp.float32),
            pltpu.VMEM((n,), jnp.float32),
        ],
    )(functools.partial(_sc_body, row0, rows // _NW))
    return f(x, score)


def kernel(x, score):
    m = x.shape[0]
    r = m - _SC_ROWS
    z = _sc_call(x, score, r, _SC_ROWS)
    y = _tc_call(x, score, r)
    return lax.dynamic_update_slice(y, z, (r, 0))


# speedup vs baseline: 3.3640x; 1.0225x over previous
import functools

import jax
import jax.numpy as jnp
from jax import lax
from jax.experimental import pallas as pl
from jax.experimental.pallas import tpu as pltpu
from jax.experimental.pallas import tpu_sc as plsc

_BLK = 8
_KEEP = 4

_N = 4096
_NW = 32
_NV = _N // 16

_SC_ROWS = 1024
_TC_TM = 3584
_TC_TN = 128


def _key2d(s):
    b = jax.lax.bitcast_convert_type(jnp.where(s == 0.0, 0.0, s), jnp.int32)
    return b ^ jax.lax.shift_right_logical(jax.lax.shift_right_arithmetic(b, 31), 1)


def _wgroll(a, k):
    n = a.shape[-1]
    idx = (jnp.arange(n) // _BLK) * _BLK + (jnp.arange(n) - k) % _BLK
    idx = jnp.broadcast_to(idx[None, :], a.shape)
    return jnp.take_along_axis(a, idx, axis=-1)


def _tc_kernel_body(x_ref, s_ref, o_ref):
    s = s_ref[...]
    x = x_ref[...]
    key = _key2d(s)
    key1 = key + 1
    lane = jax.lax.broadcasted_iota(jnp.int32, s.shape, 1) % _BLK
    rank = jnp.zeros(s.shape, jnp.float32)
    for k in range(1, 5):
        t = _wgroll(key, k)
        c = t < jnp.where(lane >= k, key1, key)
        cf = jnp.where(c, 1.0, 0.0)
        rank = rank + cf
        if k < 4:
            rank = rank - _wgroll(cf, -k)
    o_ref[...] = jnp.where(rank >= 1.0, x, 0.0)


def _tc_call(x, score, rows):
    m, n = x.shape
    grid = (rows // _TC_TM, n // _TC_TN)
    spec = pl.BlockSpec((_TC_TM, _TC_TN), lambda i, j: (i, j))
    return pl.pallas_call(
        _tc_kernel_body,
        grid=grid,
        in_specs=[spec, spec],
        out_specs=spec,
        out_shape=jax.ShapeDtypeStruct((m, n), x.dtype),
        compiler_params=pltpu.CompilerParams(
            dimension_semantics=("parallel", "parallel"),
        ),
    )(x, score)


def _sc_body(row0, rpw, x_hbm, s_hbm, o_hbm, sbuf, xbuf, obuf):
    cid = lax.axis_index("c")
    sid = lax.axis_index("s")
    wid = sid * 2 + cid

    i16 = lax.iota(jnp.int32, 16)
    lane = i16 & (_BLK - 1)
    group = i16 & ~(_BLK - 1)
    perms = [group + ((i16 - k) & (_BLK - 1)) for k in range(1, _BLK)]
    padds = [jnp.where(lane >= k, 1, 0) for k in range(1, _BLK)]

    def row_loop(r, carry):
        orow = wid * rpw + r
        irow = row0 + orow
        pltpu.sync_copy(s_hbm.at[irow], sbuf)
        pltpu.sync_copy(x_hbm.at[irow], xbuf)

        def vec_loop(j, c2):
            base = j * 16
            key = _key2d(sbuf[pl.ds(base, 16)])
            rank = jnp.zeros((16,), jnp.int32)
            for k in range(1, _BLK):
                t = key.at[perms[k - 1]].get(mode="promise_in_bounds")
                ck = t < key + padds[k - 1]
                rank = rank + jnp.where(ck, 1, 0)
            xv = xbuf[pl.ds(base, 16)]
            obuf[pl.ds(base, 16)] = jnp.where(rank >= _BLK - _KEEP, xv, 0.0)
            return c2

        lax.fori_loop(0, _NV, vec_loop, 0)
        pltpu.sync_copy(obuf, o_hbm.at[orow])
        return carry

    lax.fori_loop(0, rpw, row_loop, 0)


def _sc_call(x, score, row0, rows):
    n = x.shape[1]
    mesh = plsc.VectorSubcoreMesh(core_axis_name="c", subcore_axis_name="s")
    f = functools.partial(
        pl.kernel,
        mesh=mesh,
        out_type=jax.ShapeDtypeStruct((rows, n), jnp.float32),
        scratch_types=[
            pltpu.VMEM((n,), jnp.float32),
            pltpu.VMEM((n,), jnp.float32),
            pltpu.VMEM((n,), jnp.float32),
        ],
    )(functools.partial(_sc_body, row0, rows // _NW))
    return f(x, score)


def kernel(x, score):
    m = x.shape[0]
    r = m - _SC_ROWS
    z = _sc_call(x, score, r, _SC_ROWS)
    y = _tc_call(x, score, r)
    return lax.dynamic_update_slice(y, z, (r, 0))
